# Initial kernel scaffold; baseline (speedup 1.0000x reference)
import jax
import jax.numpy as jnp
from jax.experimental import pallas as pl


def kernel(input_ids, tok_emb, pos_emb, Wqk, Wv, Wo, ln1_g, ln1_b, W1, b1, W2, b2, ln2_g, ln2_b, fc_W):
    raise NotImplementedError("write your pallas kernel here")


# speedup vs baseline: 1.0098x; 1.0098x over previous
import functools

import jax
import jax.numpy as jnp
from jax.experimental import pallas as pl

H = 16
CHUNK = 64
NBUCKETS = 64
MAXLEN = 2048


def _layer_norm(x, g, b):
    mu = jnp.mean(x, axis=-1, keepdims=True)
    var = jnp.var(x, axis=-1, keepdims=True)
    return (x - mu) / jnp.sqrt(var + 1e-5) * g + b


def _look_back(t):
    return jnp.concatenate([jnp.roll(t, 1, axis=2), t], axis=3)


def _lsh_attention(qk, v, key):
    b, h, n, d = qk.shape
    rot = jax.random.normal(key, (d, NBUCKETS // 2), dtype=qk.dtype)
    proj = jnp.einsum('bhnd,df->bhnf', qk, rot)
    proj = jnp.concatenate([proj, -proj], axis=-1)
    buckets = jnp.argmax(proj, axis=-1)
    ticker = jnp.broadcast_to(jnp.arange(n), buckets.shape)
    s = buckets * n + ticker
    sort_idx = jnp.argsort(s, axis=-1)
    undo_idx = jnp.argsort(sort_idx, axis=-1)
    sqk = jnp.take_along_axis(qk, sort_idx[..., None], axis=2)
    sv = jnp.take_along_axis(v, sort_idx[..., None], axis=2)
    st = jnp.take_along_axis(ticker, sort_idx, axis=2)
    nc = n // CHUNK
    bq = sqk.reshape(b, h, nc, CHUNK, d)
    bk = bq / (jnp.linalg.norm(bq, axis=-1, keepdims=True) + 1e-6)
    bv = sv.reshape(b, h, nc, CHUNK, d)
    bt = st.reshape(b, h, nc, CHUNK)
    bk2 = _look_back(bk)
    bv2 = _look_back(bv)
    kt = _look_back(bt)
    dots = jnp.einsum('bhctd,bhcsd->bhcts', bq, bk2) * (d ** -0.5)
    self_mask = bt[..., :, None] == kt[..., None, :]
    dots = jnp.where(self_mask, -1e5, dots)
    attn = jax.nn.softmax(dots, axis=-1)
    bo = jnp.einsum('bhcts,bhcsd->bhctd', attn, bv2)
    so = bo.reshape(b, h, n, d)
    return jnp.take_along_axis(so, undo_idx[..., None], axis=2)


def _attn_block(x, Wqk, Wv, Wo, key):
    b, n, dim = x.shape
    dh = dim // H
    qk = (x @ Wqk).reshape(b, n, H, dh).transpose(0, 2, 1, 3)
    v = (x @ Wv).reshape(b, n, H, dh).transpose(0, 2, 1, 3)
    o = _lsh_attention(qk, v, key)
    o = o.transpose(0, 2, 1, 3).reshape(b, n, dim)
    return o @ Wo


def _pool_fc_kernel(h_ref, w_ref, o_ref):
    s = pl.program_id(0)
    hblk = h_ref[...]
    pooled = jnp.sum(hblk, axis=1) * (1.0 / MAXLEN)
    part = jax.lax.dot(pooled, w_ref[...], preferred_element_type=jnp.float32)

    @pl.when(s == 0)
    def _init():
        o_ref[...] = part

    @pl.when(s != 0)
    def _acc():
        o_ref[...] += part


def _pool_fc(h, fc_W):
    B, S, D = h.shape
    E = fc_W.shape[1]
    S_TILE = 256
    grid = (S // S_TILE,)
    return pl.pallas_call(
        _pool_fc_kernel,
        grid=grid,
        in_specs=[
            pl.BlockSpec((B, S_TILE, D), lambda s: (0, s, 0)),
            pl.BlockSpec((D, E), lambda s: (0, 0)),
        ],
        out_specs=pl.BlockSpec((B, E), lambda s: (0, 0)),
        out_shape=jax.ShapeDtypeStruct((B, E), jnp.float32),
    )(h, fc_W)


def kernel(input_ids, tok_emb, pos_emb, Wqk, Wv, Wo, ln1_g, ln1_b, W1, b1, W2, b2, ln2_g, ln2_b, fc_W):
    slen = input_ids.shape[1]
    if slen < MAXLEN:
        pad = jnp.zeros((input_ids.shape[0], MAXLEN - slen), dtype=input_ids.dtype)
        input_ids = jnp.concatenate([input_ids, pad], axis=1)
    else:
        input_ids = input_ids[:, :MAXLEN]
    x = tok_emb[input_ids] + pos_emb[None, :, :]
    rkey = jax.random.key(42)
    for i in range(Wqk.shape[0]):
        x = x + _attn_block(_layer_norm(x, ln1_g[i], ln1_b[i]), Wqk[i], Wv[i], Wo[i], jax.random.fold_in(rkey, i))
        h2 = _layer_norm(x, ln2_g[i], ln2_b[i])
        x = x + (jax.nn.gelu(h2 @ W1[i] + b1[i]) @ W2[i] + b2[i])
    return _pool_fc(x, fc_W)


# speedup vs baseline: 2.2693x; 2.2473x over previous
import functools

import jax
import jax.numpy as jnp
from jax.experimental import pallas as pl
from jax.experimental.pallas import tpu as pltpu

H = 16
DH = 64
CHUNK = 64
NBUCKETS = 64
S = 2048
D = 1024
NG = 32
BF = jnp.bfloat16


def _ln(x, g, b):
    mu = jnp.mean(x, axis=-1, keepdims=True)
    var = jnp.mean((x - mu) ** 2, axis=-1, keepdims=True)
    return (x - mu) / jnp.sqrt(var + 1e-5) * g + b


def _qkv_kernel(x_ref, g_ref, b_ref, w_ref, o_ref):
    h = _ln(x_ref[...], g_ref[...], b_ref[...])
    o_ref[...] = jax.lax.dot(h.astype(BF), w_ref[...],
                             preferred_element_type=jnp.float32)


def _qkv(x2d, g, b, w_bf):
    M = x2d.shape[0]
    MT = 512
    return pl.pallas_call(
        _qkv_kernel,
        grid=(M // MT,),
        in_specs=[
            pl.BlockSpec((MT, D), lambda i: (i, 0)),
            pl.BlockSpec((1, D), lambda i: (0, 0)),
            pl.BlockSpec((1, D), lambda i: (0, 0)),
            pl.BlockSpec((D, 2 * D), lambda i: (0, 0)),
        ],
        out_specs=pl.BlockSpec((MT, 2 * D), lambda i: (i, 0)),
        out_shape=jax.ShapeDtypeStruct((M, 2 * D), jnp.float32),
    )(x2d, g.reshape(1, D), b.reshape(1, D), w_bf)


def _route_kernel(x_ref, rot_ref, t64_ref, t32x_ref, u64x_ref, p_ref):
    rot = rot_ref[...]
    t64 = t64_ref[...]
    t32x = t32x_ref[...]
    u64x = u64x_ref[...]
    iota_b = jax.lax.broadcasted_iota(jnp.int32, (S, NBUCKETS), 1).astype(
        jnp.float32)
    for h in range(H):
        qk = x_ref[0, :, h * 128:h * 128 + DH]
        proj = jax.lax.dot(qk.astype(BF), rot,
                           preferred_element_type=jnp.float32)
        projc = jnp.concatenate([proj, -proj], axis=1)
        mx = jnp.max(projc, axis=1, keepdims=True)
        bkt = jnp.min(jnp.where(projc >= mx, iota_b, 64.0), axis=1,
                      keepdims=True)
        onehot = (bkt == iota_b).astype(BF)
        o3 = onehot.reshape(NG, CHUNK, NBUCKETS)
        c3 = jnp.einsum('ts,gsb->gtb', t64, o3,
                        preferred_element_type=jnp.float32)
        totals = c3[:, CHUNK - 1, :]
        gofs = jax.lax.dot(t32x, totals,
                           preferred_element_type=jnp.float32)
        tot_all = jnp.sum(totals, axis=0, keepdims=True)
        cum_excl = jax.lax.dot(tot_all, u64x,
                               preferred_element_type=jnp.float32)
        add = c3 - 1.0 + gofs[:, None, :] + cum_excl[0][None, None, :]
        p3 = jnp.sum(o3.astype(jnp.float32) * add, axis=2)
        p_ref[0, h] = p3.astype(jnp.int32)


def _route(qkv, rot_bf, B):
    t64 = jnp.tril(jnp.ones((CHUNK, CHUNK), BF))
    t32x = jnp.tril(jnp.ones((NG, NG), jnp.float32), -1)
    u64x = jnp.triu(jnp.ones((64, 64), jnp.float32), 1)
    return pl.pallas_call(
        _route_kernel,
        grid=(B,),
        in_specs=[
            pl.BlockSpec((1, S, 2 * D), lambda i: (i, 0, 0)),
            pl.BlockSpec((DH, 32), lambda i: (0, 0)),
            pl.BlockSpec((CHUNK, CHUNK), lambda i: (0, 0)),
            pl.BlockSpec((NG, NG), lambda i: (0, 0)),
            pl.BlockSpec((64, 64), lambda i: (0, 0)),
        ],
        out_specs=pl.BlockSpec((1, H, NG, CHUNK), lambda i: (i, 0, 0, 0)),
        out_shape=jax.ShapeDtypeStruct((B, H, NG, CHUNK), jnp.int32),
    )(qkv, rot_bf, t64, t32x, u64x)


def _attn_kernel(s_ref, o_ref, qb_ref, kb_ref, vb_ref):
    q = s_ref[0, :, 0:DH]
    v = s_ref[0, :, DH:2 * DH]
    nrm = jnp.sqrt(jnp.sum(q * q, axis=1, keepdims=True))
    k = q / (nrm + 1e-6)
    qb_ref[...] = q.astype(BF)
    kb_ref[...] = k.astype(BF)
    vb_ref[...] = v.astype(BF)
    nc = S // CHUNK
    mask = (jax.lax.broadcasted_iota(jnp.int32, (CHUNK, 2 * CHUNK), 1)
            == jax.lax.broadcasted_iota(jnp.int32, (CHUNK, 2 * CHUNK), 0) + CHUNK)
    scale = DH ** -0.5

    def body(c, _):
        pc = (c - 1) % nc
        qc = qb_ref[pl.ds(c * CHUNK, CHUNK), :]
        kcat = jnp.concatenate([kb_ref[pl.ds(pc * CHUNK, CHUNK), :],
                                kb_ref[pl.ds(c * CHUNK, CHUNK), :]], axis=0)
        dots = jax.lax.dot_general(
            qc, kcat, (((1,), (1,)), ((), ())),
            preferred_element_type=jnp.float32) * scale
        dots = jnp.where(mask, -1e5, dots)
        m = jnp.max(dots, axis=1, keepdims=True)
        e = jnp.exp(dots - m)
        attn = e / jnp.sum(e, axis=1, keepdims=True)
        vcat = jnp.concatenate([vb_ref[pl.ds(pc * CHUNK, CHUNK), :],
                                vb_ref[pl.ds(c * CHUNK, CHUNK), :]], axis=0)
        o_ref[0, pl.ds(c * CHUNK, CHUNK), :] = jax.lax.dot(
            attn.astype(BF), vcat, preferred_element_type=jnp.float32)
        return 0

    jax.lax.fori_loop(0, nc, body, 0)


def _attn(sorted_bh):
    BH = sorted_bh.shape[0]
    return pl.pallas_call(
        _attn_kernel,
        grid=(BH,),
        in_specs=[pl.BlockSpec((1, S, 2 * DH), lambda i: (i, 0, 0))],
        out_specs=pl.BlockSpec((1, S, DH), lambda i: (i, 0, 0)),
        out_shape=jax.ShapeDtypeStruct((BH, S, DH), jnp.float32),
        scratch_shapes=[pltpu.VMEM((S, DH), BF)] * 3,
    )(sorted_bh)


def _proj_res_kernel(o_ref, x_ref, w_ref, y_ref):
    y_ref[...] = x_ref[...] + jax.lax.dot(
        o_ref[...].astype(BF), w_ref[...], preferred_element_type=jnp.float32)


def _proj_res(o2d, x2d, w_bf):
    M = x2d.shape[0]
    MT = 512
    return pl.pallas_call(
        _proj_res_kernel,
        grid=(M // MT,),
        in_specs=[
            pl.BlockSpec((MT, D), lambda i: (i, 0)),
            pl.BlockSpec((MT, D), lambda i: (i, 0)),
            pl.BlockSpec((D, D), lambda i: (0, 0)),
        ],
        out_specs=pl.BlockSpec((MT, D), lambda i: (i, 0)),
        out_shape=jax.ShapeDtypeStruct((M, D), jnp.float32),
    )(o2d, x2d, w_bf)


def _ffn_kernel(x_ref, g_ref, b_ref, w1_ref, b1_ref, w2_ref, b2_ref, y_ref):
    x = x_ref[...]
    h = _ln(x, g_ref[...], b_ref[...])
    a = jax.lax.dot(h.astype(BF), w1_ref[...],
                    preferred_element_type=jnp.float32) + b1_ref[...]
    ge = jax.nn.gelu(a).astype(BF)
    y_ref[...] = x + jax.lax.dot(ge, w2_ref[...],
                                 preferred_element_type=jnp.float32) + b2_ref[...]


def _ffn(x2d, g, b, w1_bf, b1, w2_bf, b2):
    M = x2d.shape[0]
    F = w1_bf.shape[1]
    MT = 512
    return pl.pallas_call(
        _ffn_kernel,
        grid=(M // MT,),
        in_specs=[
            pl.BlockSpec((MT, D), lambda i: (i, 0)),
            pl.BlockSpec((1, D), lambda i: (0, 0)),
            pl.BlockSpec((1, D), lambda i: (0, 0)),
            pl.BlockSpec((D, F), lambda i: (0, 0)),
            pl.BlockSpec((1, F), lambda i: (0, 0)),
            pl.BlockSpec((F, D), lambda i: (0, 0)),
            pl.BlockSpec((1, D), lambda i: (0, 0)),
        ],
        out_specs=pl.BlockSpec((MT, D), lambda i: (i, 0)),
        out_shape=jax.ShapeDtypeStruct((M, D), jnp.float32),
    )(x2d, g.reshape(1, D), b.reshape(1, D), w1_bf, b1.reshape(1, F),
      w2_bf, b2.reshape(1, D))


def _pool_fc_kernel(h_ref, w_ref, o_ref):
    s = pl.program_id(0)
    pooled = jnp.sum(h_ref[...], axis=1) * (1.0 / S)
    part = jax.lax.dot(pooled.astype(BF), w_ref[...],
                       preferred_element_type=jnp.float32)

    @pl.when(s == 0)
    def _():
        o_ref[...] = part

    @pl.when(s != 0)
    def _():
        o_ref[...] += part


def _pool_fc(h3d, w_bf):
    B = h3d.shape[0]
    E = w_bf.shape[1]
    ST = 256
    return pl.pallas_call(
        _pool_fc_kernel,
        grid=(S // ST,),
        in_specs=[
            pl.BlockSpec((B, ST, D), lambda s: (0, s, 0)),
            pl.BlockSpec((D, E), lambda s: (0, 0)),
        ],
        out_specs=pl.BlockSpec((B, E), lambda s: (0, 0)),
        out_shape=jax.ShapeDtypeStruct((B, E), jnp.float32),
    )(h3d, w_bf)


def kernel(input_ids, tok_emb, pos_emb, Wqk, Wv, Wo, ln1_g, ln1_b, W1, b1, W2,
           b2, ln2_g, ln2_b, fc_W):
    B = input_ids.shape[0]
    L = Wqk.shape[0]
    slen = input_ids.shape[1]
    if slen < S:
        pad = jnp.zeros((B, S - slen), dtype=input_ids.dtype)
        input_ids = jnp.concatenate([input_ids, pad], axis=1)
    else:
        input_ids = input_ids[:, :S]

    Wq_r = Wqk.reshape(L, D, H, DH)
    Wv_r = Wv.reshape(L, D, H, DH)
    Wqkv = jnp.concatenate([Wq_r, Wv_r], axis=3).reshape(L, D, 2 * D).astype(BF)
    Wo_bf = Wo.astype(BF)
    W1_bf = W1.astype(BF)
    W2_bf = W2.astype(BF)
    fc_bf = fc_W.astype(BF)
    rkey = jax.random.key(42)
    rots = jnp.stack([
        jax.random.normal(jax.random.fold_in(rkey, i), (DH, NBUCKETS // 2),
                          dtype=jnp.float32) for i in range(L)]).astype(BF)

    x = tok_emb[input_ids] + pos_emb[None, :, :]
    bh_base = (jnp.arange(B)[:, None, None] * H
               + jnp.arange(H)[None, None, :]) * S

    for i in range(L):
        x2d = x.reshape(B * S, D)
        qkv = _qkv(x2d, ln1_g[i], ln1_b[i], Wqkv[i]).reshape(B, S, 2 * D)
        p = _route(qkv, rots[i], B)
        p_t = p.reshape(B, H, S).transpose(0, 2, 1)
        dst = (p_t + bh_base).reshape(B * S * H)
        qkv_flat = qkv.reshape(B * S * H, 2 * DH)
        sorted_flat = jnp.zeros((B * H * S, 2 * DH), jnp.float32).at[dst].set(
            qkv_flat, unique_indices=True, mode='promise_in_bounds')
        so = _attn(sorted_flat.reshape(B * H, S, 2 * DH))
        o_flat = so.reshape(B * H * S, DH)[dst]
        o2d = o_flat.reshape(B * S, D)
        x2d = _proj_res(o2d, x2d, Wo_bf[i])
        x2d = _ffn(x2d, ln2_g[i], ln2_b[i], W1_bf[i], b1[i], W2_bf[i], b2[i])
        x = x2d.reshape(B, S, D)

    return _pool_fc(x, fc_bf)


# speedup vs baseline: 3.6322x; 1.6005x over previous
import functools

import jax
import jax.numpy as jnp
from jax import lax
from jax.experimental import pallas as pl
from jax.experimental.pallas import tpu as pltpu
from jax.experimental.pallas import tpu_sc as plsc

H = 16
DH = 64
CHUNK = 64
NBUCKETS = 64
S = 2048
D = 1024
NG = 32
BF = jnp.bfloat16


def _ln(x, g, b):
    mu = jnp.mean(x, axis=-1, keepdims=True)
    var = jnp.mean((x - mu) ** 2, axis=-1, keepdims=True)
    return (x - mu) / jnp.sqrt(var + 1e-5) * g + b


def _qkv_kernel(x_ref, g_ref, b_ref, w_ref, o_ref):
    h = _ln(x_ref[...], g_ref[...], b_ref[...])
    o_ref[...] = jax.lax.dot(h.astype(BF), w_ref[...],
                             preferred_element_type=jnp.float32)


def _qkv(x2d, g, b, w_bf):
    M = x2d.shape[0]
    MT = 512
    return pl.pallas_call(
        _qkv_kernel,
        grid=(M // MT,),
        in_specs=[
            pl.BlockSpec((MT, D), lambda i: (i, 0)),
            pl.BlockSpec((1, D), lambda i: (0, 0)),
            pl.BlockSpec((1, D), lambda i: (0, 0)),
            pl.BlockSpec((D, 2 * D), lambda i: (0, 0)),
        ],
        out_specs=pl.BlockSpec((MT, 2 * D), lambda i: (i, 0)),
        out_shape=jax.ShapeDtypeStruct((M, 2 * D), jnp.float32),
    )(x2d, g.reshape(1, D), b.reshape(1, D), w_bf)


def _route_kernel(x_ref, rot_ref, t64_ref, t32x_ref, u64x_ref, p_ref):
    rot = rot_ref[...]
    t64 = t64_ref[...]
    t32x = t32x_ref[...]
    u64x = u64x_ref[...]
    iota_b = jax.lax.broadcasted_iota(jnp.int32, (S, NBUCKETS), 1).astype(
        jnp.float32)
    for h in range(H):
        qk = x_ref[0, :, h * 128:h * 128 + DH]
        proj = jax.lax.dot(qk.astype(BF), rot,
                           preferred_element_type=jnp.float32)
        projc = jnp.concatenate([proj, -proj], axis=1)
        mx = jnp.max(projc, axis=1, keepdims=True)
        bkt = jnp.min(jnp.where(projc >= mx, iota_b, 64.0), axis=1,
                      keepdims=True)
        onehot = (bkt == iota_b).astype(BF)
        o3 = onehot.reshape(NG, CHUNK, NBUCKETS)
        c3 = jnp.einsum('ts,gsb->gtb', t64, o3,
                        preferred_element_type=jnp.float32)
        totals = c3[:, CHUNK - 1, :]
        gofs = jax.lax.dot(t32x, totals,
                           preferred_element_type=jnp.float32)
        tot_all = jnp.sum(totals, axis=0, keepdims=True)
        cum_excl = jax.lax.dot(tot_all, u64x,
                               preferred_element_type=jnp.float32)
        add = c3 - 1.0 + gofs[:, None, :] + cum_excl[0][None, None, :]
        p3 = jnp.sum(o3.astype(jnp.float32) * add, axis=2)
        p_ref[0, h] = p3.astype(jnp.int32)


def _route(qkv, rot_bf, B):
    t64 = jnp.tril(jnp.ones((CHUNK, CHUNK), BF))
    t32x = jnp.tril(jnp.ones((NG, NG), jnp.float32), -1)
    u64x = jnp.triu(jnp.ones((64, 64), jnp.float32), 1)
    return pl.pallas_call(
        _route_kernel,
        grid=(B,),
        in_specs=[
            pl.BlockSpec((1, S, 2 * D), lambda i: (i, 0, 0)),
            pl.BlockSpec((DH, 32), lambda i: (0, 0)),
            pl.BlockSpec((CHUNK, CHUNK), lambda i: (0, 0)),
            pl.BlockSpec((NG, NG), lambda i: (0, 0)),
            pl.BlockSpec((64, 64), lambda i: (0, 0)),
        ],
        out_specs=pl.BlockSpec((1, H, NG, CHUNK), lambda i: (i, 0, 0, 0)),
        out_shape=jax.ShapeDtypeStruct((B, H, NG, CHUNK), jnp.int32),
    )(qkv, rot_bf, t64, t32x, u64x)


def _attn_kernel(s_ref, o_ref, qb_ref, kb_ref, vb_ref):
    q = s_ref[0, :, 0:DH]
    v = s_ref[0, :, DH:2 * DH]
    nrm = jnp.sqrt(jnp.sum(q * q, axis=1, keepdims=True))
    k = q / (nrm + 1e-6)
    qb_ref[...] = q.astype(BF)
    kb_ref[...] = k.astype(BF)
    vb_ref[...] = v.astype(BF)
    nc = S // CHUNK
    mask = (jax.lax.broadcasted_iota(jnp.int32, (CHUNK, 2 * CHUNK), 1)
            == jax.lax.broadcasted_iota(jnp.int32, (CHUNK, 2 * CHUNK), 0) + CHUNK)
    scale = DH ** -0.5

    def body(c, _):
        pc = (c - 1) % nc
        qc = qb_ref[pl.ds(c * CHUNK, CHUNK), :]
        kcat = jnp.concatenate([kb_ref[pl.ds(pc * CHUNK, CHUNK), :],
                                kb_ref[pl.ds(c * CHUNK, CHUNK), :]], axis=0)
        dots = jax.lax.dot_general(
            qc, kcat, (((1,), (1,)), ((), ())),
            preferred_element_type=jnp.float32) * scale
        dots = jnp.where(mask, -1e5, dots)
        m = jnp.max(dots, axis=1, keepdims=True)
        e = jnp.exp(dots - m)
        attn = e / jnp.sum(e, axis=1, keepdims=True)
        vcat = jnp.concatenate([vb_ref[pl.ds(pc * CHUNK, CHUNK), :],
                                vb_ref[pl.ds(c * CHUNK, CHUNK), :]], axis=0)
        res = jax.lax.dot(attn.astype(BF), vcat,
                          preferred_element_type=jnp.float32)
        o_ref[0, pl.ds(c * CHUNK, CHUNK), :] = jnp.concatenate(
            [res, jnp.zeros((CHUNK, DH), jnp.float32)], axis=1)
        return 0

    jax.lax.fori_loop(0, nc, body, 0)


def _attn(sorted_bh):
    BH = sorted_bh.shape[0]
    return pl.pallas_call(
        _attn_kernel,
        grid=(BH,),
        in_specs=[pl.BlockSpec((1, S, 2 * DH), lambda i: (i, 0, 0))],
        out_specs=pl.BlockSpec((1, S, 2 * DH), lambda i: (i, 0, 0)),
        out_shape=jax.ShapeDtypeStruct((BH, S, 2 * DH), jnp.float32),
        scratch_shapes=[pltpu.VMEM((S, DH), BF)] * 3,
    )(sorted_bh)


def _proj_res_kernel(o_ref, x_ref, w_ref, y_ref):
    y_ref[...] = x_ref[...] + jax.lax.dot(
        o_ref[...].astype(BF), w_ref[...], preferred_element_type=jnp.float32)


def _proj_res(o2d, x2d, w_bf):
    M = x2d.shape[0]
    K = w_bf.shape[0]
    MT = 512
    return pl.pallas_call(
        _proj_res_kernel,
        grid=(M // MT,),
        in_specs=[
            pl.BlockSpec((MT, K), lambda i: (i, 0)),
            pl.BlockSpec((MT, D), lambda i: (i, 0)),
            pl.BlockSpec((K, D), lambda i: (0, 0)),
        ],
        out_specs=pl.BlockSpec((MT, D), lambda i: (i, 0)),
        out_shape=jax.ShapeDtypeStruct((M, D), jnp.float32),
    )(o2d, x2d, w_bf)


def _ffn_kernel(x_ref, g_ref, b_ref, w1_ref, b1_ref, w2_ref, b2_ref, y_ref):
    x = x_ref[...]
    h = _ln(x, g_ref[...], b_ref[...])
    a = jax.lax.dot(h.astype(BF), w1_ref[...],
                    preferred_element_type=jnp.float32) + b1_ref[...]
    ge = jax.nn.gelu(a).astype(BF)
    y_ref[...] = x + jax.lax.dot(ge, w2_ref[...],
                                 preferred_element_type=jnp.float32) + b2_ref[...]


def _ffn(x2d, g, b, w1_bf, b1, w2_bf, b2):
    M = x2d.shape[0]
    F = w1_bf.shape[1]
    MT = 512
    return pl.pallas_call(
        _ffn_kernel,
        grid=(M // MT,),
        in_specs=[
            pl.BlockSpec((MT, D), lambda i: (i, 0)),
            pl.BlockSpec((1, D), lambda i: (0, 0)),
            pl.BlockSpec((1, D), lambda i: (0, 0)),
            pl.BlockSpec((D, F), lambda i: (0, 0)),
            pl.BlockSpec((1, F), lambda i: (0, 0)),
            pl.BlockSpec((F, D), lambda i: (0, 0)),
            pl.BlockSpec((1, D), lambda i: (0, 0)),
        ],
        out_specs=pl.BlockSpec((MT, D), lambda i: (i, 0)),
        out_shape=jax.ShapeDtypeStruct((M, D), jnp.float32),
    )(x2d, g.reshape(1, D), b.reshape(1, D), w1_bf, b1.reshape(1, F),
      w2_bf, b2.reshape(1, D))


def _pool_fc_kernel(h_ref, w_ref, o_ref):
    s = pl.program_id(0)
    pooled = jnp.sum(h_ref[...], axis=1) * (1.0 / S)
    part = jax.lax.dot(pooled.astype(BF), w_ref[...],
                       preferred_element_type=jnp.float32)

    @pl.when(s == 0)
    def _():
        o_ref[...] = part

    @pl.when(s != 0)
    def _():
        o_ref[...] += part


def _pool_fc(h3d, w_bf):
    B = h3d.shape[0]
    E = w_bf.shape[1]
    ST = 256
    return pl.pallas_call(
        _pool_fc_kernel,
        grid=(S // ST,),
        in_specs=[
            pl.BlockSpec((B, ST, D), lambda s: (0, s, 0)),
            pl.BlockSpec((D, E), lambda s: (0, 0)),
        ],
        out_specs=pl.BlockSpec((B, E), lambda s: (0, 0)),
        out_shape=jax.ShapeDtypeStruct((B, E), jnp.float32),
    )(h3d, w_bf)


_NW = 32
_KROW = 128


def _sc_permute(src, idx, roww, reverse):
    R = idx.shape[0]
    per_w = R // _NW
    nit = per_w // _KROW
    mesh = plsc.VectorSubcoreMesh(core_axis_name="c", subcore_axis_name="s")

    @functools.partial(
        pl.kernel, mesh=mesh,
        out_type=jax.ShapeDtypeStruct((R, roww), jnp.float32),
        scratch_types=[
            pltpu.VMEM((_KROW, roww), jnp.float32),
            pltpu.VMEM((_KROW,), jnp.int32),
            pltpu.SemaphoreType.DMA,
        ],
    )
    def k(src_hbm, idx_hbm, out_hbm, rows_v, idx_v, sem):
        wid = lax.axis_index("s") * 2 + lax.axis_index("c")
        base = wid * per_w

        def body(t, _):
            off = base + t * _KROW
            pltpu.sync_copy(idx_hbm.at[pl.ds(off, _KROW)], idx_v)
            if reverse:
                pltpu.async_copy(src_hbm.at[idx_v], rows_v, sem).wait()
                pltpu.sync_copy(rows_v, out_hbm.at[pl.ds(off, _KROW)])
            else:
                pltpu.sync_copy(src_hbm.at[pl.ds(off, _KROW)], rows_v)
                pltpu.async_copy(rows_v, out_hbm.at[idx_v], sem).wait()
            return 0

        lax.fori_loop(0, nit, body, 0)

    return k(src, idx)


def kernel(input_ids, tok_emb, pos_emb, Wqk, Wv, Wo, ln1_g, ln1_b, W1, b1, W2,
           b2, ln2_g, ln2_b, fc_W):
    B = input_ids.shape[0]
    L = Wqk.shape[0]
    slen = input_ids.shape[1]
    if slen < S:
        pad = jnp.zeros((B, S - slen), dtype=input_ids.dtype)
        input_ids = jnp.concatenate([input_ids, pad], axis=1)
    else:
        input_ids = input_ids[:, :S]

    Wq_r = Wqk.reshape(L, D, H, DH)
    Wv_r = Wv.reshape(L, D, H, DH)
    Wqkv = jnp.concatenate([Wq_r, Wv_r], axis=3).reshape(L, D, 2 * D).astype(BF)
    Wo_aug = jnp.pad(Wo.reshape(L, H, DH, D),
                     ((0, 0), (0, 0), (0, DH), (0, 0))).reshape(
                         L, 2 * D, D).astype(BF)
    W1_bf = W1.astype(BF)
    W2_bf = W2.astype(BF)
    fc_bf = fc_W.astype(BF)
    rkey = jax.random.key(42)
    rots = jnp.stack([
        jax.random.normal(jax.random.fold_in(rkey, i), (DH, NBUCKETS // 2),
                          dtype=jnp.float32) for i in range(L)]).astype(BF)

    x = tok_emb[input_ids] + pos_emb[None, :, :]
    bh_base = (jnp.arange(B)[:, None, None] * H
               + jnp.arange(H)[None, None, :]) * S

    for i in range(L):
        x2d = x.reshape(B * S, D)
        qkv = _qkv(x2d, ln1_g[i], ln1_b[i], Wqkv[i]).reshape(B, S, 2 * D)
        p = _route(qkv, rots[i], B)
        p_t = p.reshape(B, H, S).transpose(0, 2, 1)
        dst = (p_t + bh_base).reshape(B * S * H)
        qkv_flat = qkv.reshape(B * S * H, 2 * DH)
        sorted_flat = _sc_permute(qkv_flat, dst, 2 * DH, reverse=False)
        so = _attn(sorted_flat.reshape(B * H, S, 2 * DH))
        o_flat = _sc_permute(so.reshape(B * H * S, 2 * DH), dst, 2 * DH,
                             reverse=True)
        o2d = o_flat.reshape(B * S, 2 * D)
        x2d = _proj_res(o2d, x2d, Wo_aug[i])
        x2d = _ffn(x2d, ln2_g[i], ln2_b[i], W1_bf[i], b1[i], W2_bf[i], b2[i])
        x = x2d.reshape(B, S, D)

    return _pool_fc(x, fc_bf)


# speedup vs baseline: 7.1504x; 1.9687x over previous
import functools

import jax
import jax.numpy as jnp
from jax import lax
from jax.experimental import pallas as pl
from jax.experimental.pallas import tpu as pltpu
from jax.experimental.pallas import tpu_sc as plsc

H = 16
DH = 64
CHUNK = 64
NBUCKETS = 64
S = 2048
D = 1024
NG = 32
BF = jnp.bfloat16


def _ln(x, g, b):
    mu = jnp.mean(x, axis=-1, keepdims=True)
    var = jnp.mean((x - mu) ** 2, axis=-1, keepdims=True)
    return (x - mu) / jnp.sqrt(var + 1e-5) * g + b


def _qkv_kernel(x_ref, g_ref, b_ref, w_ref, o_ref):
    h = _ln(x_ref[...], g_ref[...], b_ref[...])
    o_ref[...] = jax.lax.dot(h.astype(BF), w_ref[...],
                             preferred_element_type=jnp.float32)


def _qkv(x2d, g, b, w_bf):
    M = x2d.shape[0]
    MT = 512
    return pl.pallas_call(
        _qkv_kernel,
        grid=(M // MT,),
        in_specs=[
            pl.BlockSpec((MT, D), lambda i: (i, 0)),
            pl.BlockSpec((1, D), lambda i: (0, 0)),
            pl.BlockSpec((1, D), lambda i: (0, 0)),
            pl.BlockSpec((D, 2 * D), lambda i: (0, 0)),
        ],
        out_specs=pl.BlockSpec((MT, 2 * D), lambda i: (i, 0)),
        out_shape=jax.ShapeDtypeStruct((M, 2 * D), jnp.float32),
    )(x2d, g.reshape(1, D), b.reshape(1, D), w_bf)


def _route_kernel(x_ref, rot_ref, t64_ref, t32x_ref, u64x_ref, p_ref):
    rot = rot_ref[...]
    t64 = t64_ref[...]
    t32x = t32x_ref[...]
    u64x = u64x_ref[...]
    iota_b = jax.lax.broadcasted_iota(jnp.int32, (S, NBUCKETS), 1).astype(
        jnp.float32)
    for h in range(H):
        qk = x_ref[0, :, h * 128:h * 128 + DH]
        proj = jax.lax.dot(qk.astype(BF), rot,
                           preferred_element_type=jnp.float32)
        projc = jnp.concatenate([proj, -proj], axis=1)
        mx = jnp.max(projc, axis=1, keepdims=True)
        bkt = jnp.min(jnp.where(projc >= mx, iota_b, 64.0), axis=1,
                      keepdims=True)
        onehot = (bkt == iota_b).astype(BF)
        o3 = onehot.reshape(NG, CHUNK, NBUCKETS)
        c3 = jnp.einsum('ts,gsb->gtb', t64, o3,
                        preferred_element_type=jnp.float32)
        totals = c3[:, CHUNK - 1, :]
        gofs = jax.lax.dot(t32x, totals,
                           preferred_element_type=jnp.float32)
        tot_all = jnp.sum(totals, axis=0, keepdims=True)
        cum_excl = jax.lax.dot(tot_all, u64x,
                               preferred_element_type=jnp.float32)
        add = c3 - 1.0 + gofs[:, None, :] + cum_excl[0][None, None, :]
        p3 = jnp.sum(o3.astype(jnp.float32) * add, axis=2)
        p_ref[0, h] = p3.astype(jnp.int32)


def _route(qkv, rot_bf, B):
    t64 = jnp.tril(jnp.ones((CHUNK, CHUNK), BF))
    t32x = jnp.tril(jnp.ones((NG, NG), jnp.float32), -1)
    u64x = jnp.triu(jnp.ones((64, 64), jnp.float32), 1)
    return pl.pallas_call(
        _route_kernel,
        grid=(B,),
        in_specs=[
            pl.BlockSpec((1, S, 2 * D), lambda i: (i, 0, 0)),
            pl.BlockSpec((DH, 32), lambda i: (0, 0)),
            pl.BlockSpec((CHUNK, CHUNK), lambda i: (0, 0)),
            pl.BlockSpec((NG, NG), lambda i: (0, 0)),
            pl.BlockSpec((64, 64), lambda i: (0, 0)),
        ],
        out_specs=pl.BlockSpec((1, H, NG, CHUNK), lambda i: (i, 0, 0, 0)),
        out_shape=jax.ShapeDtypeStruct((B, H, NG, CHUNK), jnp.int32),
    )(qkv, rot_bf, t64, t32x, u64x)


def _attn_kernel(s_ref, o_ref):
    nc = S // CHUNK
    q = s_ref[0, :, 0:DH]
    v = s_ref[0, :, DH:2 * DH]
    nrm = jnp.sqrt(jnp.sum(q * q, axis=1, keepdims=True))
    k = (q / (nrm + 1e-6)).astype(BF)
    vb = v.astype(BF)
    kprev = jnp.concatenate([k[S - CHUNK:, :], k[:S - CHUNK, :]], axis=0)
    vprev = jnp.concatenate([vb[S - CHUNK:, :], vb[:S - CHUNK, :]], axis=0)
    bq = q.astype(BF).reshape(nc, CHUNK, DH)
    kcat = jnp.concatenate([kprev.reshape(nc, CHUNK, DH),
                            k.reshape(nc, CHUNK, DH)], axis=1)
    vcat = jnp.concatenate([vprev.reshape(nc, CHUNK, DH),
                            vb.reshape(nc, CHUNK, DH)], axis=1)
    dots = jax.lax.dot_general(
        bq, kcat, (((2,), (2,)), ((0,), (0,))),
        preferred_element_type=jnp.float32) * (DH ** -0.5)
    mask = (jax.lax.broadcasted_iota(jnp.int32, (CHUNK, 2 * CHUNK), 1)
            == jax.lax.broadcasted_iota(jnp.int32, (CHUNK, 2 * CHUNK), 0)
            + CHUNK)
    dots = jnp.where(mask[None], -1e5, dots)
    m = jnp.max(dots, axis=2, keepdims=True)
    e = jnp.exp(dots - m)
    attn = (e / jnp.sum(e, axis=2, keepdims=True)).astype(BF)
    bo = jax.lax.dot_general(
        attn, vcat, (((2,), (1,)), ((0,), (0,))),
        preferred_element_type=jnp.float32)
    o_ref[0, :, 0:DH] = bo.reshape(S, DH)
    o_ref[0, :, DH:2 * DH] = jnp.zeros((S, DH), jnp.float32)


def _attn(sorted_bh):
    BH = sorted_bh.shape[0]
    return pl.pallas_call(
        _attn_kernel,
        grid=(BH,),
        in_specs=[pl.BlockSpec((1, S, 2 * DH), lambda i: (i, 0, 0))],
        out_specs=pl.BlockSpec((1, S, 2 * DH), lambda i: (i, 0, 0)),
        out_shape=jax.ShapeDtypeStruct((BH, S, 2 * DH), jnp.float32),
    )(sorted_bh)


def _proj_res_kernel(o_ref, x_ref, w_ref, y_ref):
    y_ref[...] = x_ref[...] + jax.lax.dot(
        o_ref[...].astype(BF), w_ref[...], preferred_element_type=jnp.float32)


def _proj_res(o2d, x2d, w_bf):
    M = x2d.shape[0]
    K = w_bf.shape[0]
    MT = 512
    return pl.pallas_call(
        _proj_res_kernel,
        grid=(M // MT,),
        in_specs=[
            pl.BlockSpec((MT, K), lambda i: (i, 0)),
            pl.BlockSpec((MT, D), lambda i: (i, 0)),
            pl.BlockSpec((K, D), lambda i: (0, 0)),
        ],
        out_specs=pl.BlockSpec((MT, D), lambda i: (i, 0)),
        out_shape=jax.ShapeDtypeStruct((M, D), jnp.float32),
    )(o2d, x2d, w_bf)


def _ffn_kernel(x_ref, g_ref, b_ref, w1_ref, b1_ref, w2_ref, b2_ref, y_ref):
    x = x_ref[...]
    h = _ln(x, g_ref[...], b_ref[...])
    a = jax.lax.dot(h.astype(BF), w1_ref[...],
                    preferred_element_type=jnp.float32) + b1_ref[...]
    ge = jax.nn.gelu(a).astype(BF)
    y_ref[...] = x + jax.lax.dot(ge, w2_ref[...],
                                 preferred_element_type=jnp.float32) + b2_ref[...]


def _ffn(x2d, g, b, w1_bf, b1, w2_bf, b2):
    M = x2d.shape[0]
    F = w1_bf.shape[1]
    MT = 512
    return pl.pallas_call(
        _ffn_kernel,
        grid=(M // MT,),
        in_specs=[
            pl.BlockSpec((MT, D), lambda i: (i, 0)),
            pl.BlockSpec((1, D), lambda i: (0, 0)),
            pl.BlockSpec((1, D), lambda i: (0, 0)),
            pl.BlockSpec((D, F), lambda i: (0, 0)),
            pl.BlockSpec((1, F), lambda i: (0, 0)),
            pl.BlockSpec((F, D), lambda i: (0, 0)),
            pl.BlockSpec((1, D), lambda i: (0, 0)),
        ],
        out_specs=pl.BlockSpec((MT, D), lambda i: (i, 0)),
        out_shape=jax.ShapeDtypeStruct((M, D), jnp.float32),
    )(x2d, g.reshape(1, D), b.reshape(1, D), w1_bf, b1.reshape(1, F),
      w2_bf, b2.reshape(1, D))


def _pool_fc_kernel(h_ref, w_ref, o_ref):
    s = pl.program_id(0)
    pooled = jnp.sum(h_ref[...], axis=1) * (1.0 / S)
    part = jax.lax.dot(pooled.astype(BF), w_ref[...],
                       preferred_element_type=jnp.float32)

    @pl.when(s == 0)
    def _():
        o_ref[...] = part

    @pl.when(s != 0)
    def _():
        o_ref[...] += part


def _pool_fc(h3d, w_bf):
    B = h3d.shape[0]
    E = w_bf.shape[1]
    ST = 256
    return pl.pallas_call(
        _pool_fc_kernel,
        grid=(S // ST,),
        in_specs=[
            pl.BlockSpec((B, ST, D), lambda s: (0, s, 0)),
            pl.BlockSpec((D, E), lambda s: (0, 0)),
        ],
        out_specs=pl.BlockSpec((B, E), lambda s: (0, 0)),
        out_shape=jax.ShapeDtypeStruct((B, E), jnp.float32),
    )(h3d, w_bf)


_NW = 32
_KROW = 128


def _sc_permute(src, idx, roww, reverse):
    R = idx.shape[0]
    per_w = R // _NW
    nit = per_w // _KROW
    mesh = plsc.VectorSubcoreMesh(core_axis_name="c", subcore_axis_name="s")

    @functools.partial(
        pl.kernel, mesh=mesh,
        out_type=jax.ShapeDtypeStruct((R, roww), jnp.float32),
        scratch_types=[
            pltpu.VMEM((2, _KROW, roww), jnp.float32),
            pltpu.VMEM((2, _KROW), jnp.int32),
            pltpu.SemaphoreType.DMA,
            pltpu.SemaphoreType.DMA,
            pltpu.SemaphoreType.DMA,
        ],
    )
    def k(src_hbm, idx_hbm, out_hbm, rows_v, idx_v, sem0, sem1, sem_o):
        wid = lax.axis_index("s") * 2 + lax.axis_index("c")
        base = wid * per_w
        sems = (sem0, sem1)

        def start_in(t, b):
            off = base + t * _KROW
            pltpu.make_async_copy(idx_hbm.at[pl.ds(off, _KROW)], idx_v.at[b],
                                  sems[b]).start()
            if not reverse:
                pltpu.make_async_copy(src_hbm.at[pl.ds(off, _KROW)],
                                      rows_v.at[b], sems[b]).start()

        def wait_in(b):
            if reverse:
                pltpu.make_async_copy(idx_hbm.at[pl.ds(0, _KROW)],
                                      idx_v.at[b], sems[b]).wait()
            else:
                pltpu.make_async_copy(idx_hbm.at[pl.ds(0, _KROW)],
                                      idx_v.at[b], sems[b]).wait()
                pltpu.make_async_copy(src_hbm.at[pl.ds(0, _KROW)],
                                      rows_v.at[b], sems[b]).wait()

        def move(t, b):
            off = base + t * _KROW
            if reverse:
                pltpu.async_copy(src_hbm.at[idx_v.at[b]], rows_v.at[b],
                                 sem_o).wait()
                pltpu.sync_copy(rows_v.at[b], out_hbm.at[pl.ds(off, _KROW)])
            else:
                pltpu.async_copy(rows_v.at[b], out_hbm.at[idx_v.at[b]],
                                 sem_o).wait()

        start_in(0, 0)

        def body(t2, _):
            t = t2 * 2
            start_in(t + 1, 1)
            wait_in(0)
            move(t, 0)

            @pl.when(t + 2 < nit)
            def _():
                start_in(t + 2, 0)

            wait_in(1)
            move(t + 1, 1)
            return 0

        lax.fori_loop(0, nit // 2, body, 0)

    return k(src, idx)


def kernel(input_ids, tok_emb, pos_emb, Wqk, Wv, Wo, ln1_g, ln1_b, W1, b1, W2,
           b2, ln2_g, ln2_b, fc_W):
    B = input_ids.shape[0]
    L = Wqk.shape[0]
    slen = input_ids.shape[1]
    if slen < S:
        pad = jnp.zeros((B, S - slen), dtype=input_ids.dtype)
        input_ids = jnp.concatenate([input_ids, pad], axis=1)
    else:
        input_ids = input_ids[:, :S]

    Wq_r = Wqk.reshape(L, D, H, DH)
    Wv_r = Wv.reshape(L, D, H, DH)
    Wqkv = jnp.concatenate([Wq_r, Wv_r], axis=3).reshape(L, D, 2 * D).astype(BF)
    Wo_aug = jnp.pad(Wo.reshape(L, H, DH, D),
                     ((0, 0), (0, 0), (0, DH), (0, 0))).reshape(
                         L, 2 * D, D).astype(BF)
    W1_bf = W1.astype(BF)
    W2_bf = W2.astype(BF)
    fc_bf = fc_W.astype(BF)
    rkey = jax.random.key(42)
    rots = jnp.stack([
        jax.random.normal(jax.random.fold_in(rkey, i), (DH, NBUCKETS // 2),
                          dtype=jnp.float32) for i in range(L)]).astype(BF)

    x = tok_emb[input_ids] + pos_emb[None, :, :]
    bh_base = (jnp.arange(B)[:, None, None] * H
               + jnp.arange(H)[None, None, :]) * S

    for i in range(L):
        x2d = x.reshape(B * S, D)
        qkv = _qkv(x2d, ln1_g[i], ln1_b[i], Wqkv[i]).reshape(B, S, 2 * D)
        p = _route(qkv, rots[i], B)
        p_t = p.reshape(B, H, S).transpose(0, 2, 1)
        dst = (p_t + bh_base).reshape(B * S * H)
        qkv_flat = qkv.reshape(B * S * H, 2 * DH)
        sorted_flat = _sc_permute(qkv_flat, dst, 2 * DH, reverse=False)
        so = _attn(sorted_flat.reshape(B * H, S, 2 * DH))
        o_flat = _sc_permute(so.reshape(B * H * S, 2 * DH), dst, 2 * DH,
                             reverse=True)
        o2d = o_flat.reshape(B * S, 2 * D)
        x2d = _proj_res(o2d, x2d, Wo_aug[i])
        x2d = _ffn(x2d, ln2_g[i], ln2_b[i], W1_bf[i], b1[i], W2_bf[i], b2[i])
        x = x2d.reshape(B, S, D)

    return _pool_fc(x, fc_bf)


# speedup vs baseline: 7.1597x; 1.0013x over previous
import functools

import jax
import jax.numpy as jnp
from jax import lax
from jax.experimental import pallas as pl
from jax.experimental.pallas import tpu as pltpu
from jax.experimental.pallas import tpu_sc as plsc

H = 16
DH = 64
CHUNK = 64
NBUCKETS = 64
S = 2048
D = 1024
NG = 32
BF = jnp.bfloat16


def _ln(x, g, b):
    mu = jnp.mean(x, axis=-1, keepdims=True)
    var = jnp.mean((x - mu) ** 2, axis=-1, keepdims=True)
    return (x - mu) / jnp.sqrt(var + 1e-5) * g + b


def _qkv_kernel(x_ref, g_ref, b_ref, w_ref, o_ref):
    h = _ln(x_ref[...], g_ref[...], b_ref[...])
    o_ref[...] = jax.lax.dot(h.astype(BF), w_ref[...],
                             preferred_element_type=jnp.float32)


def _qkv(x2d, g, b, w_bf):
    M = x2d.shape[0]
    MT = 512
    return pl.pallas_call(
        _qkv_kernel,
        grid=(M // MT,),
        in_specs=[
            pl.BlockSpec((MT, D), lambda i: (i, 0)),
            pl.BlockSpec((1, D), lambda i: (0, 0)),
            pl.BlockSpec((1, D), lambda i: (0, 0)),
            pl.BlockSpec((D, 2 * D), lambda i: (0, 0)),
        ],
        out_specs=pl.BlockSpec((MT, 2 * D), lambda i: (i, 0)),
        out_shape=jax.ShapeDtypeStruct((M, 2 * D), jnp.float32),
    )(x2d, g.reshape(1, D), b.reshape(1, D), w_bf)


def _route_kernel(x_ref, rot_ref, t64_ref, t32x_ref, u64x_ref, p_ref):
    rot = rot_ref[...]
    t64 = t64_ref[...]
    t32x = t32x_ref[...]
    u64x = u64x_ref[...]
    iota_b = jax.lax.broadcasted_iota(jnp.int32, (S, NBUCKETS), 1).astype(
        jnp.float32)
    for h in range(H):
        qk = x_ref[0, :, h * 128:h * 128 + DH]
        proj = jax.lax.dot(qk.astype(BF), rot,
                           preferred_element_type=jnp.float32)
        projc = jnp.concatenate([proj, -proj], axis=1)
        mx = jnp.max(projc, axis=1, keepdims=True)
        bkt = jnp.min(jnp.where(projc >= mx, iota_b, 64.0), axis=1,
                      keepdims=True)
        onehot = (bkt == iota_b).astype(BF)
        o3 = onehot.reshape(NG, CHUNK, NBUCKETS)
        c3 = jnp.einsum('ts,gsb->gtb', t64, o3,
                        preferred_element_type=jnp.float32)
        totals = c3[:, CHUNK - 1, :]
        gofs = jax.lax.dot(t32x, totals,
                           preferred_element_type=jnp.float32)
        tot_all = jnp.sum(totals, axis=0, keepdims=True)
        cum_excl = jax.lax.dot(tot_all, u64x,
                               preferred_element_type=jnp.float32)
        add = c3 - 1.0 + gofs[:, None, :] + cum_excl[0][None, None, :]
        p3 = jnp.sum(o3.astype(jnp.float32) * add, axis=2)
        p_ref[0, h] = p3.astype(jnp.int32)


def _route(qkv, rot_bf, B):
    t64 = jnp.tril(jnp.ones((CHUNK, CHUNK), BF))
    t32x = jnp.tril(jnp.ones((NG, NG), jnp.float32), -1)
    u64x = jnp.triu(jnp.ones((64, 64), jnp.float32), 1)
    return pl.pallas_call(
        _route_kernel,
        grid=(B,),
        in_specs=[
            pl.BlockSpec((1, S, 2 * D), lambda i: (i, 0, 0)),
            pl.BlockSpec((DH, 32), lambda i: (0, 0)),
            pl.BlockSpec((CHUNK, CHUNK), lambda i: (0, 0)),
            pl.BlockSpec((NG, NG), lambda i: (0, 0)),
            pl.BlockSpec((64, 64), lambda i: (0, 0)),
        ],
        out_specs=pl.BlockSpec((1, H, NG, CHUNK), lambda i: (i, 0, 0, 0)),
        out_shape=jax.ShapeDtypeStruct((B, H, NG, CHUNK), jnp.int32),
    )(qkv, rot_bf, t64, t32x, u64x)


def _attn_kernel(s_ref, o_ref):
    nc = S // CHUNK
    q = s_ref[0, :, 0:DH]
    v = s_ref[0, :, DH:2 * DH]
    nrm = jnp.sqrt(jnp.sum(q * q, axis=1, keepdims=True))
    k = (q / (nrm + 1e-6)).astype(BF)
    vb = v.astype(BF)
    kprev = jnp.concatenate([k[S - CHUNK:, :], k[:S - CHUNK, :]], axis=0)
    vprev = jnp.concatenate([vb[S - CHUNK:, :], vb[:S - CHUNK, :]], axis=0)
    bq = q.astype(BF).reshape(nc, CHUNK, DH)
    kcat = jnp.concatenate([kprev.reshape(nc, CHUNK, DH),
                            k.reshape(nc, CHUNK, DH)], axis=1)
    vcat = jnp.concatenate([vprev.reshape(nc, CHUNK, DH),
                            vb.reshape(nc, CHUNK, DH)], axis=1)
    dots = jax.lax.dot_general(
        bq, kcat, (((2,), (2,)), ((0,), (0,))),
        preferred_element_type=jnp.float32) * (DH ** -0.5)
    mask = (jax.lax.broadcasted_iota(jnp.int32, (CHUNK, 2 * CHUNK), 1)
            == jax.lax.broadcasted_iota(jnp.int32, (CHUNK, 2 * CHUNK), 0)
            + CHUNK)
    dots = jnp.where(mask[None], -1e5, dots)
    m = jnp.max(dots, axis=2, keepdims=True)
    e = jnp.exp(dots - m)
    attn = (e / jnp.sum(e, axis=2, keepdims=True)).astype(BF)
    bo = jax.lax.dot_general(
        attn, vcat, (((2,), (1,)), ((0,), (0,))),
        preferred_element_type=jnp.float32)
    o_ref[0, :, 0:DH] = bo.reshape(S, DH)
    o_ref[0, :, DH:2 * DH] = jnp.zeros((S, DH), jnp.float32)


def _attn(sorted_bh):
    BH = sorted_bh.shape[0]
    return pl.pallas_call(
        _attn_kernel,
        grid=(BH,),
        in_specs=[pl.BlockSpec((1, S, 2 * DH), lambda i: (i, 0, 0))],
        out_specs=pl.BlockSpec((1, S, 2 * DH), lambda i: (i, 0, 0)),
        out_shape=jax.ShapeDtypeStruct((BH, S, 2 * DH), jnp.float32),
    )(sorted_bh)


def _proj_res_kernel(o_ref, x_ref, w_ref, y_ref):
    y_ref[...] = x_ref[...] + jax.lax.dot(
        o_ref[...].astype(BF), w_ref[...], preferred_element_type=jnp.float32)


def _proj_res(o2d, x2d, w_bf):
    M = x2d.shape[0]
    K = w_bf.shape[0]
    MT = 512
    return pl.pallas_call(
        _proj_res_kernel,
        grid=(M // MT,),
        in_specs=[
            pl.BlockSpec((MT, K), lambda i: (i, 0)),
            pl.BlockSpec((MT, D), lambda i: (i, 0)),
            pl.BlockSpec((K, D), lambda i: (0, 0)),
        ],
        out_specs=pl.BlockSpec((MT, D), lambda i: (i, 0)),
        out_shape=jax.ShapeDtypeStruct((M, D), jnp.float32),
    )(o2d, x2d, w_bf)


def _ffn_kernel(x_ref, g_ref, b_ref, w1_ref, b1_ref, w2_ref, b2_ref, y_ref):
    x = x_ref[...]
    h = _ln(x, g_ref[...], b_ref[...])
    a = jax.lax.dot(h.astype(BF), w1_ref[...],
                    preferred_element_type=jnp.float32) + b1_ref[...]
    ge = jax.nn.gelu(a).astype(BF)
    y_ref[...] = x + jax.lax.dot(ge, w2_ref[...],
                                 preferred_element_type=jnp.float32) + b2_ref[...]


def _ffn(x2d, g, b, w1_bf, b1, w2_bf, b2):
    M = x2d.shape[0]
    F = w1_bf.shape[1]
    MT = 512
    return pl.pallas_call(
        _ffn_kernel,
        grid=(M // MT,),
        in_specs=[
            pl.BlockSpec((MT, D), lambda i: (i, 0)),
            pl.BlockSpec((1, D), lambda i: (0, 0)),
            pl.BlockSpec((1, D), lambda i: (0, 0)),
            pl.BlockSpec((D, F), lambda i: (0, 0)),
            pl.BlockSpec((1, F), lambda i: (0, 0)),
            pl.BlockSpec((F, D), lambda i: (0, 0)),
            pl.BlockSpec((1, D), lambda i: (0, 0)),
        ],
        out_specs=pl.BlockSpec((MT, D), lambda i: (i, 0)),
        out_shape=jax.ShapeDtypeStruct((M, D), jnp.float32),
    )(x2d, g.reshape(1, D), b.reshape(1, D), w1_bf, b1.reshape(1, F),
      w2_bf, b2.reshape(1, D))


def _pool_fc_kernel(h_ref, w_ref, o_ref):
    s = pl.program_id(0)
    pooled = jnp.sum(h_ref[...], axis=1) * (1.0 / S)
    part = jax.lax.dot(pooled.astype(BF), w_ref[...],
                       preferred_element_type=jnp.float32)

    @pl.when(s == 0)
    def _():
        o_ref[...] = part

    @pl.when(s != 0)
    def _():
        o_ref[...] += part


def _pool_fc(h3d, w_bf):
    B = h3d.shape[0]
    E = w_bf.shape[1]
    ST = 256
    return pl.pallas_call(
        _pool_fc_kernel,
        grid=(S // ST,),
        in_specs=[
            pl.BlockSpec((B, ST, D), lambda s: (0, s, 0)),
            pl.BlockSpec((D, E), lambda s: (0, 0)),
        ],
        out_specs=pl.BlockSpec((B, E), lambda s: (0, 0)),
        out_shape=jax.ShapeDtypeStruct((B, E), jnp.float32),
    )(h3d, w_bf)


_NW = 32
_KROW = 128


def _sc_permute(src, idx, roww, reverse):
    R = idx.shape[0]
    per_w = R // _NW
    nit = per_w // _KROW
    mesh = plsc.VectorSubcoreMesh(core_axis_name="c", subcore_axis_name="s")

    @functools.partial(
        pl.kernel, mesh=mesh,
        out_type=jax.ShapeDtypeStruct((R, roww), jnp.float32),
        scratch_types=[
            pltpu.VMEM((2, _KROW, roww), jnp.float32),
            pltpu.VMEM((2, _KROW), jnp.int32),
            pltpu.SemaphoreType.DMA,
            pltpu.SemaphoreType.DMA,
            pltpu.SemaphoreType.DMA,
        ],
    )
    def k(src_hbm, idx_hbm, out_hbm, rows_v, idx_v, sem0, sem1, sem_o):
        wid = lax.axis_index("s") * 2 + lax.axis_index("c")
        base = wid * per_w
        sems = (sem0, sem1)

        def start_in(t, b):
            off = base + t * _KROW
            pltpu.make_async_copy(idx_hbm.at[pl.ds(off, _KROW)], idx_v.at[b],
                                  sems[b]).start()
            if not reverse:
                pltpu.make_async_copy(src_hbm.at[pl.ds(off, _KROW)],
                                      rows_v.at[b], sems[b]).start()

        def wait_in(b):
            pltpu.make_async_copy(idx_hbm.at[pl.ds(0, _KROW)],
                                  idx_v.at[b], sems[b]).wait()
            if not reverse:
                pltpu.make_async_copy(src_hbm.at[pl.ds(0, _KROW)],
                                      rows_v.at[b], sems[b]).wait()

        def move(t, b):
            off = base + t * _KROW
            if reverse:
                pltpu.async_copy(src_hbm.at[idx_v.at[b]], rows_v.at[b],
                                 sem_o).wait()
                pltpu.sync_copy(rows_v.at[b], out_hbm.at[pl.ds(off, _KROW)])
            else:
                pltpu.async_copy(rows_v.at[b], out_hbm.at[idx_v.at[b]],
                                 sem_o).wait()

        start_in(0, 0)

        def body(t2, _):
            t = t2 * 2
            start_in(t + 1, 1)
            wait_in(0)
            move(t, 0)

            @pl.when(t + 2 < nit)
            def _():
                start_in(t + 2, 0)

            wait_in(1)
            move(t + 1, 1)
            return 0

        lax.fori_loop(0, nit // 2, body, 0)

    return k(src, idx)


def kernel(input_ids, tok_emb, pos_emb, Wqk, Wv, Wo, ln1_g, ln1_b, W1, b1, W2,
           b2, ln2_g, ln2_b, fc_W):
    B = input_ids.shape[0]
    L = Wqk.shape[0]
    slen = input_ids.shape[1]
    if slen < S:
        pad = jnp.zeros((B, S - slen), dtype=input_ids.dtype)
        input_ids = jnp.concatenate([input_ids, pad], axis=1)
    else:
        input_ids = input_ids[:, :S]

    Wq_r = Wqk.reshape(L, D, H, DH)
    Wv_r = Wv.reshape(L, D, H, DH)
    Wqkv = jnp.concatenate([Wq_r, Wv_r], axis=3).reshape(L, D, 2 * D).astype(BF)
    Wo_aug = jnp.pad(Wo.reshape(L, H, DH, D),
                     ((0, 0), (0, 0), (0, DH), (0, 0))).reshape(
                         L, 2 * D, D).astype(BF)
    W1_bf = W1.astype(BF)
    W2_bf = W2.astype(BF)
    fc_bf = fc_W.astype(BF)
    rkey = jax.random.key(42)
    rots = jnp.stack([
        jax.random.normal(jax.random.fold_in(rkey, i), (DH, NBUCKETS // 2),
                          dtype=jnp.float32) for i in range(L)]).astype(BF)

    x = tok_emb[input_ids] + pos_emb[None, :, :]

    BG = 1
    Bg = B // BG
    bh_base = (jnp.arange(Bg)[:, None, None] * H
               + jnp.arange(H)[None, None, :]) * S
    xs = [x[g * Bg:(g + 1) * Bg] for g in range(BG)]

    for i in range(L):
        for g in range(BG):
            x2d = xs[g].reshape(Bg * S, D)
            qkv = _qkv(x2d, ln1_g[i], ln1_b[i], Wqkv[i]).reshape(Bg, S, 2 * D)
            p = _route(qkv, rots[i], Bg)
            p_t = p.reshape(Bg, H, S).transpose(0, 2, 1)
            dst = (p_t + bh_base).reshape(Bg * S * H)
            qkv_flat = qkv.reshape(Bg * S * H, 2 * DH)
            sorted_flat = _sc_permute(qkv_flat, dst, 2 * DH, reverse=False)
            so = _attn(sorted_flat.reshape(Bg * H, S, 2 * DH))
            o_flat = _sc_permute(so.reshape(Bg * H * S, 2 * DH), dst, 2 * DH,
                                 reverse=True)
            o2d = o_flat.reshape(Bg * S, 2 * D)
            x2d = _proj_res(o2d, x2d, Wo_aug[i])
            x2d = _ffn(x2d, ln2_g[i], ln2_b[i], W1_bf[i], b1[i], W2_bf[i],
                       b2[i])
            xs[g] = x2d.reshape(Bg, S, D)

    return _pool_fc(jnp.concatenate(xs, axis=0), fc_bf)
